# Initial kernel scaffold; baseline (speedup 1.0000x reference)
import jax
import jax.numpy as jnp
from jax.experimental import pallas as pl


def kernel(weather, week, W_weather, W_week, fc_W, fc_b):
    raise NotImplementedError("write your pallas kernel here")


# speedup vs baseline: 6.5327x; 6.5327x over previous
import functools

import jax
import jax.numpy as jnp
from jax import lax
from jax.experimental import pallas as pl
from jax.experimental.pallas import tpu as pltpu
from jax.experimental.pallas import tpu_sc as plsc

EMBED = 64
OUT = 64
LANES = 16

_NC = 2
_NS = 16
_NW = _NC * _NS

_TOTAL = 4096 * 200
_PER_W = _TOTAL // _NW
_IDXROW = 128
_ROWS_PER_CHUNK = 4
_CHUNK = _IDXROW * _ROWS_PER_CHUNK
_NCHUNKS = _PER_W // _CHUNK


def _tables_body(wW_ref, wK_ref, fcW_ref, fcb_ref, tW_ref, tK_ref):
    fw = fcW_ref[...]
    tW_ref[...] = jnp.dot(wW_ref[...], fw[0:EMBED, :],
                          preferred_element_type=jnp.float32)
    tK_ref[...] = jnp.dot(wK_ref[...], fw[EMBED:, :],
                          preferred_element_type=jnp.float32) + fcb_ref[...]


_tables = pl.pallas_call(
    _tables_body,
    out_shape=(
        jax.ShapeDtypeStruct((1000, EMBED), jnp.float32),
        jax.ShapeDtypeStruct((1000, EMBED), jnp.float32),
    ),
)


@functools.partial(
    pl.kernel,
    mesh=plsc.VectorSubcoreMesh(core_axis_name="c", subcore_axis_name="s"),
    compiler_params=pltpu.CompilerParams(use_tc_tiling_on_sc=False),
    out_type=jax.ShapeDtypeStruct((_TOTAL, OUT), jnp.float32),
    scratch_types=[
        pltpu.VMEM((_ROWS_PER_CHUNK, _IDXROW), jnp.int32),
        pltpu.VMEM((_ROWS_PER_CHUNK, _IDXROW), jnp.int32),
        pltpu.VMEM((_CHUNK, OUT), jnp.float32),
        pltpu.VMEM((_CHUNK, OUT), jnp.float32),
        pltpu.SemaphoreType.DMA,
        pltpu.SemaphoreType.DMA,
    ],
)
def _sc_gather_add(tW_hbm, tK_hbm, wthr_hbm, week_hbm, out_hbm,
                   idx_a, idx_b, rows_a, rows_b, sem_a, sem_b):
    wid = lax.axis_index("s") * _NC + lax.axis_index("c")
    idx_base = wid * (_PER_W // _IDXROW)
    out_base = wid * _PER_W

    def chunk_body(ci, carry):
        pltpu.sync_copy(wthr_hbm.at[pl.ds(idx_base + ci * _ROWS_PER_CHUNK,
                                          _ROWS_PER_CHUNK)], idx_a)
        pltpu.sync_copy(week_hbm.at[pl.ds(idx_base + ci * _ROWS_PER_CHUNK,
                                          _ROWS_PER_CHUNK)], idx_b)
        copies = []
        for j in range(_ROWS_PER_CHUNK):
            dst = pl.ds(j * _IDXROW, _IDXROW)
            copies.append(pltpu.async_copy(
                tW_hbm.at[idx_a.at[j]], rows_a.at[dst], sem_a))
            copies.append(pltpu.async_copy(
                tK_hbm.at[idx_b.at[j]], rows_b.at[dst], sem_b))
        for c in copies:
            c.wait()

        def add_body(i, acc):
            for d in range(OUT // LANES):
                sl = pl.ds(d * LANES, LANES)
                rows_a[i, sl] = rows_a[i, sl] + rows_b[i, sl]
            return acc

        lax.fori_loop(0, _CHUNK, add_body, 0)
        pltpu.sync_copy(rows_a,
                        out_hbm.at[pl.ds(out_base + ci * _CHUNK, _CHUNK)])
        return carry

    lax.fori_loop(0, _NCHUNKS, chunk_body, 0)


def kernel(weather, week, W_weather, W_week, fc_W, fc_b):
    tW, tK = _tables(W_weather, W_week, fc_W, fc_b.reshape(1, OUT))
    wthr = weather.astype(jnp.int32).reshape(-1, _IDXROW)
    wk = week.astype(jnp.int32).reshape(-1, _IDXROW)
    out = _sc_gather_add(tW, tK, wthr, wk)
    return out.reshape(weather.shape[0], weather.shape[1], OUT)
